# baseline (speedup 1.0000x reference)
import functools

import jax
import jax.numpy as jnp
from jax.experimental import pallas as pl
from jax.experimental.pallas import tpu as pltpu

_CHUNK = 8192
_TOPK = 5
_BIG_I32 = 2**30
_NEG_INF = float("-inf")


def _scan_kernel(q_ref, m_ref, qual_ref, s_out, i_out, run_s, run_i, *, n_rows, n_chunks):
    i = pl.program_id(0)

    @pl.when(i == 0)
    def _init():
        run_s[...] = jnp.full(run_s.shape, _NEG_INF, jnp.float32)
        run_i[...] = jnp.full(run_i.shape, _BIG_I32, jnp.int32)

    m = m_ref[...]
    qual = qual_ref[...]
    rowsq = jnp.sum(m * m, axis=1, keepdims=True)
    scale = qual / jnp.maximum(jnp.sqrt(rowsq), 1e-12)
    ms = m * scale

    q = q_ref[...]
    qn = q / jnp.maximum(jnp.sqrt(jnp.sum(q * q, axis=1, keepdims=True)), 1e-12)

    sims = jnp.dot(qn, ms.T, preferred_element_type=jnp.float32)

    base = i * _CHUNK
    nq = sims.shape[0]
    idx2d = jax.lax.broadcasted_iota(jnp.int32, (nq, _CHUNK), 1)
    sims = jnp.where(base + idx2d < n_rows, sims, _NEG_INF)

    mx0 = jnp.max(sims, axis=1, keepdims=True)
    thresh = run_s[:, _TOPK - 1:_TOPK]
    trig = jnp.any(mx0 > thresh)

    @pl.when(trig)
    def _update():
        s = sims
        sc_cols = []
        ix_cols = []
        for j in range(_TOPK):
            mx = mx0 if j == 0 else jnp.max(s, axis=1, keepdims=True)
            am = jnp.min(jnp.where(s == mx, idx2d, _BIG_I32), axis=1, keepdims=True)
            sc_cols.append(mx)
            ix_cols.append(base + am)
            if j < _TOPK - 1:
                s = jnp.where(idx2d == am, _NEG_INF, s)
        cs = jnp.concatenate(sc_cols, axis=1)
        ci = jnp.concatenate(ix_cols, axis=1)

        s10 = jnp.concatenate([run_s[...], cs], axis=1)
        i10 = jnp.concatenate([run_i[...], ci], axis=1)
        ns_cols = []
        ni_cols = []
        for j in range(_TOPK):
            mx = jnp.max(s10, axis=1, keepdims=True)
            am = jnp.min(jnp.where(s10 == mx, i10, _BIG_I32), axis=1, keepdims=True)
            ns_cols.append(mx)
            ni_cols.append(am)
            if j < _TOPK - 1:
                s10 = jnp.where(i10 == am, _NEG_INF, s10)
        run_s[...] = jnp.concatenate(ns_cols, axis=1)
        run_i[...] = jnp.concatenate(ni_cols, axis=1)

    @pl.when(i == n_chunks - 1)
    def _finish():
        s_out[...] = run_s[...]
        i_out[...] = run_i[...]


def _gather_kernel(idx_ref, m0, m1, m2, m3, m4, s_ref, x_ref, o_ref):
    q = pl.program_id(0)
    rows = jnp.concatenate([m0[0], m1[0], m2[0], m3[0], m4[0]], axis=0)
    srow = s_ref[pl.ds(q, 1), :]
    srow = srow - jnp.max(srow, axis=1, keepdims=True)
    e = jnp.exp(srow)
    w = e / jnp.sum(e, axis=1, keepdims=True)
    retrieved = jnp.dot(w, rows, preferred_element_type=jnp.float32)
    xrow = x_ref[pl.ds(q, 1), :]
    o_ref[0] = xrow + 0.5 * retrieved


def _run(q, memory_mean, memory_quality):
    n_rows, c = memory_mean.shape
    nq = q.shape[0]
    n_chunks = pl.cdiv(n_rows, _CHUNK)

    scores, idxs = pl.pallas_call(
        functools.partial(_scan_kernel, n_rows=n_rows, n_chunks=n_chunks),
        grid=(n_chunks,),
        in_specs=[
            pl.BlockSpec((nq, c), lambda i: (0, 0)),
            pl.BlockSpec((_CHUNK, c), lambda i: (i, 0)),
            pl.BlockSpec((_CHUNK, 1), lambda i: (i, 0)),
        ],
        out_specs=[
            pl.BlockSpec((nq, _TOPK), lambda i: (0, 0)),
            pl.BlockSpec((nq, _TOPK), lambda i: (0, 0)),
        ],
        out_shape=[
            jax.ShapeDtypeStruct((nq, _TOPK), jnp.float32),
            jax.ShapeDtypeStruct((nq, _TOPK), jnp.int32),
        ],
        scratch_shapes=[
            pltpu.VMEM((nq, _TOPK), jnp.float32),
            pltpu.VMEM((nq, _TOPK), jnp.int32),
        ],
    )(q, memory_mean, memory_quality.reshape(n_rows, 1))

    m3d = memory_mean.reshape(n_rows, 1, c)
    idx_flat = idxs.reshape(-1)

    def mk_mspec(k):
        return pl.BlockSpec((1, 1, c), lambda qq, idx_ref: (idx_ref[qq * _TOPK + k], 0, 0))

    out = pl.pallas_call(
        _gather_kernel,
        grid_spec=pltpu.PrefetchScalarGridSpec(
            num_scalar_prefetch=1,
            grid=(nq,),
            in_specs=[mk_mspec(0), mk_mspec(1), mk_mspec(2), mk_mspec(3), mk_mspec(4),
                      pl.BlockSpec((nq, _TOPK), lambda qq, idx_ref: (0, 0)),
                      pl.BlockSpec((nq, c), lambda qq, idx_ref: (0, 0))],
            out_specs=pl.BlockSpec((1, 1, c), lambda qq, idx_ref: (qq, 0, 0)),
        ),
        out_shape=jax.ShapeDtypeStruct((nq, 1, c), jnp.float32),
    )(idx_flat, m3d, m3d, m3d, m3d, m3d, scores, q)

    return out.reshape(nq, c)


def kernel(x, memory_mean, memory_quality):
    b, s, c = x.shape
    q = x.reshape(b * s, c)
    out = _run(q, memory_mean, memory_quality)
    return out.reshape(b, s, c)
